# baseline (speedup 1.0000x reference)
import jax
import jax.numpy as jnp
from jax import lax
from jax.experimental import pallas as pl
from jax.experimental.pallas import tpu as pltpu
from jax.experimental.pallas import tpu_sc as plsc

N_GENES = 1000000
N_DRUGS = 1000
EMB_DIM = 64
BATCH = 16384
HALF = N_GENES // 2

NC = 2
NS = 16
LANES = 16
NW = NC * NS
B_PER_W = BATCH // NW
IDX_CHUNK = 128
N_CHUNKS = B_PER_W // IDX_CHUNK
D_VREGS = EMB_DIM // LANES


def _body(h1_hbm, h2_hbm, drug_hbm, gi_hbm, giA_hbm, giB_hbm, di_hbm, w_hbm,
          out_hbm, gidx_v, idxA_v, idxB_v, didx_v,
          growsA, growsB, drows, w_v, pscr, out_v, sem):
    wid = lax.axis_index("s") * NC + lax.axis_index("c")
    base = wid * B_PER_W

    pltpu.sync_copy(gi_hbm.at[wid], gidx_v)
    pltpu.sync_copy(giA_hbm.at[wid], idxA_v)
    pltpu.sync_copy(giB_hbm.at[wid], idxB_v)
    pltpu.sync_copy(di_hbm.at[wid], didx_v)
    pltpu.sync_copy(w_hbm, w_v)

    copies = []
    for c in range(N_CHUNKS):
        copies.append(pltpu.async_copy(
            h1_hbm.at[idxA_v.at[c]],
            growsA.at[pl.ds(c * IDX_CHUNK, IDX_CHUNK)], sem))
        copies.append(pltpu.async_copy(
            h2_hbm.at[idxB_v.at[c]],
            growsB.at[pl.ds(c * IDX_CHUNK, IDX_CHUNK)], sem))
        copies.append(pltpu.async_copy(
            drug_hbm.at[didx_v.at[c]],
            drows.at[pl.ds(c * IDX_CHUNK, IDX_CHUNK)], sem))
    for cp in copies:
        cp.wait()

    wregs = [w_v[pl.ds(c * LANES, LANES)] for c in range(D_VREGS)]
    iota = lax.broadcasted_iota(jnp.int32, (LANES,), 0)
    colbase = iota * LANES

    def group_body(g, carry):
        rowbase = g * LANES
        gvec = gidx_v[pl.ds(rowbase, LANES)]
        for r in range(LANES):
            row = rowbase + r
            in_lo = gvec[r] < HALF
            acc = None
            for c in range(D_VREGS):
                gva = growsA[row, pl.ds(c * LANES, LANES)]
                gvb = growsB[row, pl.ds(c * LANES, LANES)]
                gv = jnp.where(in_lo, gva, gvb)
                dv = drows[row, pl.ds(c * LANES, LANES)]
                t = gv * dv * wregs[c]
                acc = t if acc is None else acc + t
            pscr[pl.ds(r * LANES, LANES)] = acc
        cv = colbase
        tot = plsc.load_gather(pscr, [cv])
        for _ in range(LANES - 1):
            cv = cv + 1
            tot = tot + plsc.load_gather(pscr, [cv])
        out_v[pl.ds(rowbase, LANES)] = tot
        return carry

    lax.fori_loop(0, B_PER_W // LANES, group_body, 0)
    pltpu.sync_copy(out_v, out_hbm.at[pl.ds(base, B_PER_W)])


def _dist_mult_sc(h1, h2, drug_emb, gi, giA, giB, di, w):
    mesh = plsc.VectorSubcoreMesh(core_axis_name="c", subcore_axis_name="s",
                                  num_cores=NC, num_subcores=NS)
    return pl.kernel(
        _body,
        out_type=jax.ShapeDtypeStruct((BATCH,), jnp.float32),
        mesh=mesh,
        compiler_params=pltpu.CompilerParams(needs_layout_passes=False,
                                             use_tc_tiling_on_sc=False),
        scratch_types=[
            pltpu.VMEM((B_PER_W,), jnp.int32),
            pltpu.VMEM((N_CHUNKS, IDX_CHUNK), jnp.int32),
            pltpu.VMEM((N_CHUNKS, IDX_CHUNK), jnp.int32),
            pltpu.VMEM((N_CHUNKS, IDX_CHUNK), jnp.int32),
            pltpu.VMEM((B_PER_W, EMB_DIM), jnp.float32),
            pltpu.VMEM((B_PER_W, EMB_DIM), jnp.float32),
            pltpu.VMEM((B_PER_W, EMB_DIM), jnp.float32),
            pltpu.VMEM((EMB_DIM,), jnp.float32),
            pltpu.VMEM((LANES * LANES,), jnp.float32),
            pltpu.VMEM((B_PER_W,), jnp.float32),
            pltpu.SemaphoreType.DMA,
        ],
    )(h1, h2, drug_emb, gi, giA, giB, di, w)


def kernel(gene_idx, drug_idx, gene_emb, drug_emb, W):
    h1 = gene_emb[:HALF]
    h2 = gene_emb[HALF:]
    gi32 = gene_idx.astype(jnp.int32)
    gi = gi32.reshape(NW, B_PER_W)
    giA = jnp.minimum(gi32, HALF - 1).reshape(NW, N_CHUNKS, IDX_CHUNK)
    giB = jnp.clip(gi32 - HALF, 0, HALF - 1).reshape(NW, N_CHUNKS, IDX_CHUNK)
    di = drug_idx.astype(jnp.int32).reshape(NW, N_CHUNKS, IDX_CHUNK)
    return _dist_mult_sc(h1, h2, drug_emb, gi, giA, giB, di, W)
